# baseline (speedup 1.0000x reference)
import jax
import jax.numpy as jnp
from jax.experimental import pallas as pl
from jax.experimental.pallas import tpu as pltpu

N, C, H, O = 512, 64, 128, 64
NBITS = 7
TJ = 8
TP = TJ // 2
NSTEPS = N // TJ

_NAMES = ('op0', 'red0', 'exp1', 'op1', 'red1', 'exp2', 'op2')


def _body(*refs):
    (gates, x0, x1, rows), rest = refs[:4], refs[4:]
    pairs = rest[:TP]
    w = dict(zip(
        [n + s for n in _NAMES for s in ('_W1', '_b1', '_W2', '_b2')],
        rest[TP:TP + 28]))
    wb_lo, wb_hi = rest[TP + 28:TP + 30]
    out0, out1, out2, af, bfac, mxp, mnp = rest[TP + 30:]

    jb = pl.program_id(0)
    f32 = jnp.float32

    def g(k):
        return gates[k:k + 1, :O]

    def mlp(x, name):
        h = jnp.maximum(
            jnp.dot(x, w[name + '_W1'][...], preferred_element_type=f32)
            + w[name + '_b1'][...], 0.0)
        return (jnp.dot(h, w[name + '_W2'][...], preferred_element_type=f32)
                + w[name + '_b2'][...])

    @pl.when(jb == 0)
    def _():
        x1f = x1[...]
        af[...] = jnp.dot(x1f, w['exp2_W1'][0:C, :], preferred_element_type=f32)
        bfac[...] = jnp.dot(x1f, w['exp2_W1'][C:2 * C, :], preferred_element_type=f32)
        mxp[...] = jnp.zeros((N, 2 * C), f32)
        mnp[...] = jnp.ones((N, 2 * C), f32)
        r1 = jnp.concatenate([jnp.max(x1f, axis=0, keepdims=True),
                              jnp.min(x1f, axis=0, keepdims=True)], axis=-1)
        s0 = mlp(x0[...], 'op0') * g(0) + mlp(r1, 'red0') * g(1)
        out0[...] = jax.nn.sigmoid(s0) * g(7)

    rows_flat = rows[...].reshape(TJ * N, C)
    rm = jnp.dot(rows_flat, w['op2_W1'][0:C, :], preferred_element_type=f32)
    pair_vals = [pairs[p][...] for p in range(TP)]
    cms = []
    for p in range(TP):
        cms.append(jnp.dot(pair_vals[p], wb_lo[...], preferred_element_type=f32))
        cms.append(jnp.dot(pair_vals[p], wb_hi[...], preferred_element_type=f32))
    cm = jnp.concatenate(cms, axis=0)
    h2 = jnp.maximum(rm + cm + w['op2_b1'][...], 0.0)
    a_j = af[pl.ds(jb * TJ, TJ), :]
    he = jnp.maximum(
        (a_j[:, None, :] + bfac[...][None, :, :]).reshape(TJ * N, H)
        + w['exp2_b1'][...], 0.0)
    s2 = ((jnp.dot(h2, w['op2_W2'][...], preferred_element_type=f32)
           + w['op2_b2'][...]) * g(6)
          + (jnp.dot(he, w['exp2_W2'][...], preferred_element_type=f32)
             + w['exp2_b2'][...]) * g(5))
    out2[...] = (jax.nn.sigmoid(s2) * g(9)).reshape(TJ, N, O)

    rid = jax.lax.broadcasted_iota(jnp.int32, (N, 2 * C), 0)
    li = jax.lax.broadcasted_iota(jnp.int32, (N, 2 * C), 1)
    mxv, mnv = mxp[...], mnp[...]
    for p in range(TP):
        j0 = jb * TJ + 2 * p
        dmask = ((li < C) & (rid == j0)) | ((li >= C) & (rid == (j0 + 1)))
        mxv = jnp.maximum(mxv, jnp.where(dmask, 0.0, pair_vals[p]))
        mnv = jnp.minimum(mnv, jnp.where(dmask, 1.0, pair_vals[p]))
    mxp[...] = mxv
    mnp[...] = mnv

    @pl.when(jb == NSTEPS - 1)
    def _():
        mxf = jnp.maximum(mxp[:, 0:C], mxp[:, C:2 * C])
        mnf = jnp.minimum(mnp[:, 0:C], mnp[:, C:2 * C])
        red = jnp.concatenate([mxf, mnf], axis=-1)
        s1 = (mlp(red, 'red1') * g(4) + mlp(x1[...], 'op1') * g(3)
              + mlp(x0[...], 'exp1') * g(2))
        out1[...] = jax.nn.sigmoid(s1) * g(8)


def kernel(x0, x1, x2, params, action):
    f32 = jnp.float32
    x1s = x1.reshape(N, C)
    x2s = x2.reshape(N, N, C)
    x2c = x2.reshape(N, N * C)

    a = jnp.asarray(action, jnp.int32)
    bfs = [((a >> (NBITS - 1 - k)) & 1).astype(f32) for k in range(NBITS)]
    act0 = (bfs[0] + bfs[1] > 0).astype(f32)
    act1 = (bfs[2] + bfs[3] + bfs[4] > 0).astype(f32)
    act2 = (bfs[5] + bfs[6] > 0).astype(f32)
    gvec = jnp.stack(bfs + [act0, act1, act2] + [jnp.zeros(())] * 6)
    gates = jnp.broadcast_to(gvec[:, None], (16, 128)).astype(f32)

    weights = []
    wspecs = []
    for name in _NAMES:
        for suff in ('_W1', '_b1', '_W2', '_b2'):
            wgt = params[name + suff]
            if wgt.ndim == 1:
                wgt = wgt.reshape(1, -1)
            weights.append(wgt)
            wspecs.append(pl.BlockSpec(wgt.shape, lambda jb: (0, 0)))

    wb = params['op2_W1'][C:2 * C, :]
    zpad = jnp.zeros((C, H), f32)
    wb_lo = jnp.concatenate([wb, zpad], axis=0)
    wb_hi = jnp.concatenate([zpad, wb], axis=0)
    weights += [wb_lo, wb_hi]
    wspecs += [pl.BlockSpec((2 * C, H), lambda jb: (0, 0))] * 2

    pair_specs = [
        pl.BlockSpec((N, 2 * C), lambda jb, p=p: (0, jb * TP + p))
        for p in range(TP)
    ]

    out0, out1, out2 = pl.pallas_call(
        _body,
        grid=(NSTEPS,),
        in_specs=[
            pl.BlockSpec((16, 128), lambda jb: (0, 0)),
            pl.BlockSpec((1, C), lambda jb: (0, 0)),
            pl.BlockSpec((N, C), lambda jb: (0, 0)),
            pl.BlockSpec((TJ, N, C), lambda jb: (jb, 0, 0)),
        ] + pair_specs + wspecs,
        out_specs=[
            pl.BlockSpec((1, O), lambda jb: (0, 0)),
            pl.BlockSpec((N, O), lambda jb: (0, 0)),
            pl.BlockSpec((TJ, N, O), lambda jb: (jb, 0, 0)),
        ],
        out_shape=[
            jax.ShapeDtypeStruct((1, O), f32),
            jax.ShapeDtypeStruct((N, O), f32),
            jax.ShapeDtypeStruct((N, N, O), f32),
        ],
        scratch_shapes=[
            pltpu.VMEM((N, H), f32),
            pltpu.VMEM((N, H), f32),
            pltpu.VMEM((N, 2 * C), f32),
            pltpu.VMEM((N, 2 * C), f32),
        ],
        compiler_params=pltpu.CompilerParams(
            dimension_semantics=("arbitrary",),
        ),
    )(gates, x0, x1s, x2s, *([x2c] * TP), *weights)

    return out0, out1.reshape(1, N, O), out2.reshape(1, N, N, O)
